# Initial kernel scaffold; baseline (speedup 1.0000x reference)
import jax
import jax.numpy as jnp
from jax.experimental import pallas as pl


def kernel(x, token_types, W1s, b1s, W2s, b2s, W1l, b1l, W2l, b2l):
    raise NotImplementedError("write your pallas kernel here")


# speedup vs baseline: 2.6453x; 2.6453x over previous
import functools

import jax
import jax.numpy as jnp
from jax import lax
from jax.experimental import pallas as pl
from jax.experimental.pallas import tpu as pltpu
from jax.experimental.pallas import tpu_sc as plsc

IN_F = 1024
HID_F = 4096
OUT_F = 1024
NTOK = 8192
T = 256
S = NTOK + T
NB = S // T
NW = 32


def _make_row_gather(n_out, d, ch):
    per_w = n_out // NW
    nch = per_w // ch
    assert per_w % ch == 0 and per_w % 8 == 0 and ch % 8 == 0 and ch <= 128
    mesh = plsc.VectorSubcoreMesh(core_axis_name="c", subcore_axis_name="s")

    @functools.partial(
        pl.kernel,
        mesh=mesh,
        out_type=jax.ShapeDtypeStruct((n_out, d), jnp.float32),
        scratch_types=[
            pltpu.VMEM((per_w,), jnp.int32),
            pltpu.VMEM((ch, d), jnp.float32),
            pltpu.VMEM((ch, d), jnp.float32),
            pltpu.SemaphoreType.DMA,
            pltpu.SemaphoreType.DMA,
        ],
    )
    def gather_k(table_hbm, idx_hbm, out_hbm, idx_v, buf0, buf1, sem0, sem1):
        wid = lax.axis_index("s") * 2 + lax.axis_index("c")
        base = wid * per_w
        pltpu.sync_copy(idx_hbm.at[pl.ds(base, per_w)], idx_v)
        bufs = (buf0, buf1)
        sems = (sem0, sem1)

        def start(c, b):
            pltpu.async_copy(
                table_hbm.at[idx_v.at[pl.ds(c * ch, ch)]], bufs[b], sems[b]
            )

        def drain(c, b):
            pltpu.make_async_copy(
                table_hbm.at[idx_v.at[pl.ds(c * ch, ch)]], bufs[b], sems[b]
            ).wait()
            pltpu.sync_copy(bufs[b], out_hbm.at[pl.ds(base + c * ch, ch)])

        start(0, 0)

        def step(g, _):
            @pl.when(2 * g + 1 < nch)
            def _():
                start(2 * g + 1, 1)

            drain(2 * g, 0)

            @pl.when(2 * g + 1 < nch)
            def _():
                @pl.when(2 * g + 2 < nch)
                def _():
                    start(2 * g + 2, 0)

                drain(2 * g + 1, 1)

        lax.fori_loop(0, (nch + 1) // 2, step, None)

    return gather_k


_gather_dispatch = _make_row_gather(S, IN_F, 24)
_gather_assemble = _make_row_gather(NTOK, OUT_F, 32)


def _mlp_body(eid_ref, x_ref, w1_ref, b1_ref, w2_ref, b2_ref, o_ref):
    del eid_ref
    xb = x_ref[...].astype(jnp.bfloat16)
    h = lax.dot_general(
        xb, w1_ref[0], (((1,), (1,)), ((), ())),
        preferred_element_type=jnp.float32,
    )
    h = h + b1_ref[0, 0, :][None, :]
    h = (0.5 * h * (1.0 + lax.erf(h * 0.7071067811865476))).astype(jnp.bfloat16)
    o = lax.dot_general(
        h, w2_ref[0], (((1,), (1,)), ((), ())),
        preferred_element_type=jnp.float32,
    )
    o_ref[...] = o + b2_ref[0, 0, :][None, :]


def _mlp_blocks(eid, xs, w1, b1, w2, b2):
    grid_spec = pltpu.PrefetchScalarGridSpec(
        num_scalar_prefetch=1,
        grid=(NB,),
        in_specs=[
            pl.BlockSpec((T, IN_F), lambda i, e: (i, 0)),
            pl.BlockSpec((1, HID_F, IN_F), lambda i, e: (e[i], 0, 0)),
            pl.BlockSpec((1, 1, HID_F), lambda i, e: (e[i], 0, 0)),
            pl.BlockSpec((1, OUT_F, HID_F), lambda i, e: (e[i], 0, 0)),
            pl.BlockSpec((1, 1, OUT_F), lambda i, e: (e[i], 0, 0)),
        ],
        out_specs=pl.BlockSpec((T, OUT_F), lambda i, e: (i, 0)),
    )
    return pl.pallas_call(
        _mlp_body,
        grid_spec=grid_spec,
        out_shape=jax.ShapeDtypeStruct((S, OUT_F), jnp.float32),
    )(eid, xs, w1, b1, w2, b2)


def kernel(x, token_types, W1s, b1s, W2s, b2s, W1l, b1l, W2l, b2l):
    Bv, Nv, C = x.shape
    x_flat = x.reshape(NTOK, C)
    tt = token_types.reshape(NTOK).astype(jnp.int32)

    m0 = (tt == 0).astype(jnp.int32)
    c0 = jnp.cumsum(m0)
    n0 = c0[NTOK - 1]
    rank0 = c0 - m0
    m1 = 1 - m0
    rank1 = jnp.cumsum(m1) - m1
    n0p = ((n0 + T - 1) // T) * T
    dst = jnp.where(m0 == 1, rank0, n0p + rank1)
    src = jnp.zeros((S,), jnp.int32).at[dst].set(
        jnp.arange(NTOK, dtype=jnp.int32)
    )
    eid = (jnp.arange(NB, dtype=jnp.int32) * T >= n0p).astype(jnp.int32)

    w1 = jnp.stack([W1s, W1l]).astype(jnp.bfloat16)
    b1 = jnp.stack([b1s, b1l]).reshape(2, 1, HID_F)
    w2 = jnp.stack([W2s, W2l]).astype(jnp.bfloat16)
    b2 = jnp.stack([b2s, b2l]).reshape(2, 1, OUT_F)

    xs = _gather_dispatch(x_flat, src)
    ys = _mlp_blocks(eid, xs, w1, b1, w2, b2)
    out = _gather_assemble(ys, dst)
    return out.reshape(Bv, Nv, C)
